# baseline (speedup 1.0000x reference)
import functools

import jax
import jax.numpy as jnp
from jax import lax
from jax.experimental import pallas as pl
from jax.experimental.pallas import tpu as pltpu

_DOWN_RATIO = 4.0
_K = 100
_BIG = 2**30


def _decode_body(hm_ref, wh_ref, reg_ref, boxes_ref, scores_ref, cls_ref,
                 s_ref, m2_ref, q_ref, *, C, H, W, K):
    h = hm_ref[0]
    heat = jnp.clip(jax.nn.sigmoid(h), 1e-4, 1.0 - 1e-4)

    neg = jnp.float32(-1.0)
    pad_w = jnp.full((C, H, 1), neg, jnp.float32)
    left = jnp.concatenate([pad_w, heat[:, :, : W - 1]], axis=2)
    right = jnp.concatenate([heat[:, :, 1:], pad_w], axis=2)
    hw = jnp.maximum(jnp.maximum(left, right), heat)
    pad_h = jnp.full((C, 1, W), neg, jnp.float32)
    up = jnp.concatenate([pad_h, hw[:, : H - 1, :]], axis=1)
    down = jnp.concatenate([hw[:, 1:, :], pad_h], axis=1)
    hmax = jnp.maximum(jnp.maximum(up, down), hw)

    sup = jnp.where(heat == hmax, heat, 0.0)
    s_ref[...] = sup
    m2_ref[...] = jnp.concatenate(
        [jnp.max(sup, axis=(1, 2))[None],
         jnp.full((1, 128 - C), neg, jnp.float32)], axis=1)
    q_ref[...] = jnp.zeros_like(q_ref)

    lane = lax.broadcasted_iota(jnp.int32, (1, 128), 1)
    flat2 = (lax.broadcasted_iota(jnp.int32, (H, W), 0) * W
             + lax.broadcasted_iota(jnp.int32, (H, W), 1))

    def roll_tree(v1, i1, v2, shifts, axis):
        for s in shifts:
            bv = pltpu.roll(v1, s, axis)
            bi = pltpu.roll(i1, s, axis)
            cond = (v1 > bv) | ((v1 == bv) & (i1 <= bi))
            if v2 is not None:
                b2 = pltpu.roll(v2, s, axis)
                v2 = jnp.maximum(jnp.minimum(v1, bv),
                                 jnp.where(cond, v2, b2))
            v1 = jnp.where(cond, v1, bv)
            i1 = jnp.where(cond, i1, bi)
        return v1, i1, v2

    def body(k, carry):
        m2 = m2_ref[...]
        cv_f, cc, _ = roll_tree(m2, lane, None, (64, 32, 16, 8, 4, 2, 1), 1)
        c = cc[0, 0]

        slab = s_ref[c]
        h = H
        a1, ai = slab, flat2
        cond = a1[: h // 2] >= a1[h // 2:]
        v2 = jnp.minimum(a1[: h // 2], a1[h // 2:])
        v1 = jnp.where(cond, a1[: h // 2], a1[h // 2:])
        i1 = jnp.where(cond, ai[: h // 2], ai[h // 2:])
        h //= 2
        while h > 8:
            h2 = h // 2
            cond = ((v1[:h2] > v1[h2:])
                    | ((v1[:h2] == v1[h2:]) & (i1[:h2] <= i1[h2:])))
            nv2 = jnp.maximum(jnp.minimum(v1[:h2], v1[h2:]),
                              jnp.where(cond, v2[:h2], v2[h2:]))
            i1 = jnp.where(cond, i1[:h2], i1[h2:])
            v1 = jnp.where(cond, v1[:h2], v1[h2:])
            v2 = nv2
            h = h2
        v1, i1, v2 = roll_tree(v1, i1, v2, (4, 2, 1), 0)
        v1, i1, v2 = roll_tree(v1, i1, v2, (64, 32, 16, 8, 4, 2, 1), 1)
        mv = v1[0:1, :]
        pos = i1[0:1, :]
        cmax = v2[0:1, :]

        posM = jnp.broadcast_to(pos, (H, W))
        s_ref[c] = jnp.where(flat2 == posM, neg, slab)
        m2_ref[...] = jnp.where(lane == cc, cmax, m2)

        sel = lane == k
        q_ref[pl.ds(0, 1), :] = jnp.where(sel, mv, q_ref[pl.ds(0, 1), :])
        q_ref[pl.ds(1, 1), :] = jnp.where(sel, cc.astype(jnp.float32),
                                          q_ref[pl.ds(1, 1), :])
        q_ref[pl.ds(2, 1), :] = jnp.where(sel, pos.astype(jnp.float32),
                                          q_ref[pl.ds(2, 1), :])
        return carry

    lax.fori_loop(0, K, body, 0, unroll=False)

    q = q_ref[...]
    score = q[0:1, :K]
    clsv = q[1:2, :K]
    posi = q[2:3, :].astype(jnp.int32)
    yi = posi // W
    xi = posi % W

    onehot_y = (lax.broadcasted_iota(jnp.int32, (H, 128), 0)
                == jnp.broadcast_to(yi, (H, 128))).astype(jnp.float32)
    onehot_x = (lax.broadcasted_iota(jnp.int32, (W, 128), 0)
                == jnp.broadcast_to(xi, (W, 128))).astype(jnp.float32)

    def gather2(arr):
        t = lax.dot_general(
            arr, onehot_y, (((0,), (0,)), ((), ())),
            precision=lax.Precision.HIGHEST,
            preferred_element_type=jnp.float32)
        return jnp.sum(t * onehot_x, axis=0, keepdims=True)

    g_reg0 = gather2(reg_ref[0, 0])
    g_reg1 = gather2(reg_ref[0, 1])
    g_wh0 = gather2(wh_ref[0, 0])
    g_wh1 = gather2(wh_ref[0, 1])

    ys = yi.astype(jnp.float32)[:, :K] + g_reg1[:, :K]
    xs = xi.astype(jnp.float32)[:, :K] + g_reg0[:, :K]
    wv = g_wh0[:, :K]
    hv = g_wh1[:, :K]
    x1 = (xs - wv * 0.5) * _DOWN_RATIO
    y1 = (ys - hv * 0.5) * _DOWN_RATIO
    x2 = (xs + wv * 0.5) * _DOWN_RATIO
    y2 = (ys + hv * 0.5) * _DOWN_RATIO
    boxes_ref[...] = jnp.concatenate([x1, y1, x2, y2], axis=0)[None]
    scores_ref[...] = score[None]
    cls_ref[...] = clsv[None]


def kernel(hm, wh, reg):
    B, C, H, W = hm.shape
    K = _K
    body = functools.partial(_decode_body, C=C, H=H, W=W, K=K)
    boxes_t, scores, classes = pl.pallas_call(
        body,
        grid=(B,),
        in_specs=[
            pl.BlockSpec((1, C, H, W), lambda b: (b, 0, 0, 0)),
            pl.BlockSpec((1, 2, H, W), lambda b: (b, 0, 0, 0)),
            pl.BlockSpec((1, 2, H, W), lambda b: (b, 0, 0, 0)),
        ],
        out_specs=[
            pl.BlockSpec((1, 4, K), lambda b: (b, 0, 0)),
            pl.BlockSpec((1, 1, K), lambda b: (b, 0, 0)),
            pl.BlockSpec((1, 1, K), lambda b: (b, 0, 0)),
        ],
        out_shape=[
            jax.ShapeDtypeStruct((B, 4, K), jnp.float32),
            jax.ShapeDtypeStruct((B, 1, K), jnp.float32),
            jax.ShapeDtypeStruct((B, 1, K), jnp.float32),
        ],
        scratch_shapes=[
            pltpu.VMEM((C, H, W), jnp.float32),
            pltpu.VMEM((1, 128), jnp.float32),
            pltpu.VMEM((8, 128), jnp.float32),
        ],
    )(hm, wh, reg)
    boxes = jnp.transpose(boxes_t, (0, 2, 1))
    return boxes, scores[:, 0, :], classes[:, 0, :]


# speedup vs baseline: 1.7137x; 1.7137x over previous
import functools

import jax
import jax.numpy as jnp
from jax import lax
from jax.experimental import pallas as pl
from jax.experimental.pallas import tpu as pltpu

_DOWN_RATIO = 4.0
_K = 100
_BIG = 2**30
_G = 2


def _decode_body(hm_ref, wh_ref, reg_ref, boxes_ref, scores_ref, cls_ref,
                 s_ref, m2_ref, q_ref, *, C, H, W, K, G):
    h = hm_ref[...].reshape(G * C, H, W)
    heat = jnp.clip(jax.nn.sigmoid(h), 1e-4, 1.0 - 1e-4)

    neg = jnp.float32(-1.0)
    pad_w = jnp.full((G * C, H, 1), neg, jnp.float32)
    left = jnp.concatenate([pad_w, heat[:, :, : W - 1]], axis=2)
    right = jnp.concatenate([heat[:, :, 1:], pad_w], axis=2)
    hw = jnp.maximum(jnp.maximum(left, right), heat)
    pad_h = jnp.full((G * C, 1, W), neg, jnp.float32)
    up = jnp.concatenate([pad_h, hw[:, : H - 1, :]], axis=1)
    down = jnp.concatenate([hw[:, 1:, :], pad_h], axis=1)
    hmax = jnp.maximum(jnp.maximum(up, down), hw)

    sup = jnp.where(heat == hmax, heat, 0.0)
    s_ref[...] = sup
    m2 = jnp.max(sup, axis=(1, 2)).reshape(G, C)
    m2_ref[...] = jnp.concatenate(
        [m2, jnp.full((G, 128 - C), neg, jnp.float32)], axis=1)
    q_ref[...] = jnp.zeros_like(q_ref)

    lane = lax.broadcasted_iota(jnp.int32, (1, 128), 1)
    flat2 = (lax.broadcasted_iota(jnp.int32, (H, W), 0) * W
             + lax.broadcasted_iota(jnp.int32, (H, W), 1))

    def body(k, carry):
        for n in range(G):
            m2n = m2_ref[pl.ds(n, 1), :]
            mv = jnp.max(m2n, axis=(0, 1), keepdims=True)
            cv = jnp.min(jnp.where(m2n == mv, lane, _BIG), axis=(0, 1),
                         keepdims=True)
            c = n * C + cv[0, 0]

            slab = s_ref[c]
            posv = jnp.min(jnp.where(slab == mv, flat2, _BIG), axis=(0, 1),
                           keepdims=True)
            new_slab = jnp.where(flat2 == posv, neg, slab)
            s_ref[c] = new_slab
            cmaxv = jnp.max(new_slab, axis=(0, 1), keepdims=True)
            m2_ref[pl.ds(n, 1), :] = jnp.where(lane == cv, cmaxv, m2n)

            vec3 = jnp.concatenate([
                mv, cv.astype(jnp.float32), posv.astype(jnp.float32),
            ], axis=0)
            q3 = q_ref[n, pl.ds(0, 3), :]
            q_ref[n, pl.ds(0, 3), :] = jnp.where(lane == k, vec3, q3)
        return carry

    lax.fori_loop(0, K, body, 0, unroll=False)

    for n in range(G):
        q = q_ref[n]
        score = q[0:1, :K]
        clsv = q[1:2, :K]
        posi = q[2:3, :].astype(jnp.int32)
        yi = posi // W
        xi = posi % W

        onehot_y = (lax.broadcasted_iota(jnp.int32, (H, 128), 0)
                    == jnp.broadcast_to(yi, (H, 128))).astype(jnp.float32)
        onehot_x = (lax.broadcasted_iota(jnp.int32, (W, 128), 0)
                    == jnp.broadcast_to(xi, (W, 128))).astype(jnp.float32)

        def gather2(arr):
            t = lax.dot_general(
                arr, onehot_y, (((0,), (0,)), ((), ())),
                precision=lax.Precision.HIGHEST,
                preferred_element_type=jnp.float32)
            return jnp.sum(t * onehot_x, axis=0, keepdims=True)

        g_reg0 = gather2(reg_ref[n, 0])
        g_reg1 = gather2(reg_ref[n, 1])
        g_wh0 = gather2(wh_ref[n, 0])
        g_wh1 = gather2(wh_ref[n, 1])

        ys = yi.astype(jnp.float32)[:, :K] + g_reg1[:, :K]
        xs = xi.astype(jnp.float32)[:, :K] + g_reg0[:, :K]
        wv = g_wh0[:, :K]
        hv = g_wh1[:, :K]
        x1 = (xs - wv * 0.5) * _DOWN_RATIO
        y1 = (ys - hv * 0.5) * _DOWN_RATIO
        x2 = (xs + wv * 0.5) * _DOWN_RATIO
        y2 = (ys + hv * 0.5) * _DOWN_RATIO
        boxes_ref[n] = jnp.concatenate([x1, y1, x2, y2], axis=0)
        scores_ref[n] = score
        cls_ref[n] = clsv


def kernel(hm, wh, reg):
    B, C, H, W = hm.shape
    K = _K
    G = _G
    body = functools.partial(_decode_body, C=C, H=H, W=W, K=K, G=G)
    boxes_t, scores, classes = pl.pallas_call(
        body,
        grid=(B // G,),
        in_specs=[
            pl.BlockSpec((G, C, H, W), lambda b: (b, 0, 0, 0)),
            pl.BlockSpec((G, 2, H, W), lambda b: (b, 0, 0, 0)),
            pl.BlockSpec((G, 2, H, W), lambda b: (b, 0, 0, 0)),
        ],
        out_specs=[
            pl.BlockSpec((G, 4, K), lambda b: (b, 0, 0)),
            pl.BlockSpec((G, 1, K), lambda b: (b, 0, 0)),
            pl.BlockSpec((G, 1, K), lambda b: (b, 0, 0)),
        ],
        out_shape=[
            jax.ShapeDtypeStruct((B, 4, K), jnp.float32),
            jax.ShapeDtypeStruct((B, 1, K), jnp.float32),
            jax.ShapeDtypeStruct((B, 1, K), jnp.float32),
        ],
        scratch_shapes=[
            pltpu.VMEM((G * C, H, W), jnp.float32),
            pltpu.VMEM((G, 128), jnp.float32),
            pltpu.VMEM((G, 8, 128), jnp.float32),
        ],
    )(hm, wh, reg)
    boxes = jnp.transpose(boxes_t, (0, 2, 1))
    return boxes, scores[:, 0, :], classes[:, 0, :]


# speedup vs baseline: 1.7166x; 1.0017x over previous
import functools

import jax
import jax.numpy as jnp
from jax import lax
from jax.experimental import pallas as pl
from jax.experimental.pallas import tpu as pltpu

_DOWN_RATIO = 4.0
_K = 100
_BIG = 2**30
_G = 2


def _decode_body(hm_ref, wh_ref, reg_ref, boxes_ref, scores_ref, cls_ref,
                 *refs, C, H, W, K, G):
    s_refs = refs[0:G]
    m2_refs = refs[G:2 * G]
    q_refs = refs[2 * G:3 * G]
    h = hm_ref[...].reshape(G * C, H, W)
    heat = jnp.clip(jax.nn.sigmoid(h), 1e-4, 1.0 - 1e-4)

    neg = jnp.float32(-1.0)
    pad_w = jnp.full((G * C, H, 1), neg, jnp.float32)
    left = jnp.concatenate([pad_w, heat[:, :, : W - 1]], axis=2)
    right = jnp.concatenate([heat[:, :, 1:], pad_w], axis=2)
    hw = jnp.maximum(jnp.maximum(left, right), heat)
    pad_h = jnp.full((G * C, 1, W), neg, jnp.float32)
    up = jnp.concatenate([pad_h, hw[:, : H - 1, :]], axis=1)
    down = jnp.concatenate([hw[:, 1:, :], pad_h], axis=1)
    hmax = jnp.maximum(jnp.maximum(up, down), hw)

    sup = jnp.where(heat == hmax, heat, 0.0)
    m2 = jnp.max(sup, axis=(1, 2)).reshape(G, C)
    pad_m2 = jnp.full((1, 128 - C), neg, jnp.float32)
    for n in range(G):
        s_refs[n][...] = sup[n * C:(n + 1) * C]
        m2_refs[n][...] = jnp.concatenate([m2[n:n + 1], pad_m2], axis=1)
        q_refs[n][...] = jnp.zeros_like(q_refs[n])

    lane = lax.broadcasted_iota(jnp.int32, (1, 128), 1)
    flat2 = (lax.broadcasted_iota(jnp.int32, (H, W), 0) * W
             + lax.broadcasted_iota(jnp.int32, (H, W), 1))

    def body(k, carry):
        for n in range(G):
            m2n = m2_refs[n][...]
            mv = jnp.max(m2n, axis=(0, 1), keepdims=True)
            cv = jnp.min(jnp.where(m2n == mv, lane, _BIG), axis=(0, 1),
                         keepdims=True)
            c = cv[0, 0]

            slab = s_refs[n][c]
            posv = jnp.min(jnp.where(slab == mv, flat2, _BIG), axis=(0, 1),
                           keepdims=True)
            new_slab = jnp.where(flat2 == posv, neg, slab)
            s_refs[n][c] = new_slab
            cmaxv = jnp.max(new_slab, axis=(0, 1), keepdims=True)
            m2_refs[n][...] = jnp.where(lane == cv, cmaxv, m2n)

            vec3 = jnp.concatenate([
                mv, cv.astype(jnp.float32), posv.astype(jnp.float32),
            ], axis=0)
            q3 = q_refs[n][pl.ds(0, 3), :]
            q_refs[n][pl.ds(0, 3), :] = jnp.where(lane == k, vec3, q3)
        return carry

    lax.fori_loop(0, K, body, 0, unroll=False)

    for n in range(G):
        q = q_refs[n][...]
        score = q[0:1, :K]
        clsv = q[1:2, :K]
        posi = q[2:3, :].astype(jnp.int32)
        yi = posi // W
        xi = posi % W

        onehot_y = (lax.broadcasted_iota(jnp.int32, (H, 128), 0)
                    == jnp.broadcast_to(yi, (H, 128))).astype(jnp.float32)
        onehot_x = (lax.broadcasted_iota(jnp.int32, (W, 128), 0)
                    == jnp.broadcast_to(xi, (W, 128))).astype(jnp.float32)

        def gather2(arr):
            t = lax.dot_general(
                arr, onehot_y, (((0,), (0,)), ((), ())),
                precision=lax.Precision.HIGHEST,
                preferred_element_type=jnp.float32)
            return jnp.sum(t * onehot_x, axis=0, keepdims=True)

        g_reg0 = gather2(reg_ref[n, 0])
        g_reg1 = gather2(reg_ref[n, 1])
        g_wh0 = gather2(wh_ref[n, 0])
        g_wh1 = gather2(wh_ref[n, 1])

        ys = yi.astype(jnp.float32)[:, :K] + g_reg1[:, :K]
        xs = xi.astype(jnp.float32)[:, :K] + g_reg0[:, :K]
        wv = g_wh0[:, :K]
        hv = g_wh1[:, :K]
        x1 = (xs - wv * 0.5) * _DOWN_RATIO
        y1 = (ys - hv * 0.5) * _DOWN_RATIO
        x2 = (xs + wv * 0.5) * _DOWN_RATIO
        y2 = (ys + hv * 0.5) * _DOWN_RATIO
        boxes_ref[n] = jnp.concatenate([x1, y1, x2, y2], axis=0)
        scores_ref[n] = score
        cls_ref[n] = clsv


def kernel(hm, wh, reg):
    B, C, H, W = hm.shape
    K = _K
    G = _G
    body = functools.partial(_decode_body, C=C, H=H, W=W, K=K, G=G)
    boxes_t, scores, classes = pl.pallas_call(
        body,
        grid=(B // G,),
        in_specs=[
            pl.BlockSpec((G, C, H, W), lambda b: (b, 0, 0, 0)),
            pl.BlockSpec((G, 2, H, W), lambda b: (b, 0, 0, 0)),
            pl.BlockSpec((G, 2, H, W), lambda b: (b, 0, 0, 0)),
        ],
        out_specs=[
            pl.BlockSpec((G, 4, K), lambda b: (b, 0, 0)),
            pl.BlockSpec((G, 1, K), lambda b: (b, 0, 0)),
            pl.BlockSpec((G, 1, K), lambda b: (b, 0, 0)),
        ],
        out_shape=[
            jax.ShapeDtypeStruct((B, 4, K), jnp.float32),
            jax.ShapeDtypeStruct((B, 1, K), jnp.float32),
            jax.ShapeDtypeStruct((B, 1, K), jnp.float32),
        ],
        scratch_shapes=(
            [pltpu.VMEM((C, H, W), jnp.float32) for _ in range(G)]
            + [pltpu.VMEM((1, 128), jnp.float32) for _ in range(G)]
            + [pltpu.VMEM((8, 128), jnp.float32) for _ in range(G)]
        ),
    )(hm, wh, reg)
    boxes = jnp.transpose(boxes_t, (0, 2, 1))
    return boxes, scores[:, 0, :], classes[:, 0, :]


# speedup vs baseline: 2.3067x; 1.3437x over previous
import functools

import jax
import jax.numpy as jnp
from jax import lax
from jax.experimental import pallas as pl
from jax.experimental.pallas import tpu as pltpu

_DOWN_RATIO = 4.0
_K = 100
_BIG = 2**30
_G = 2


def _decode_body(hm_ref, wh_ref, reg_ref, boxes_ref, scores_ref, cls_ref,
                 *refs, C, H, W, K, G):
    s_refs = refs[0:G]
    m2_refs = refs[G:2 * G]
    q_refs = refs[2 * G:3 * G]
    h = hm_ref[...].reshape(G * C, H, W)
    heat = jnp.clip(jax.nn.sigmoid(h), 1e-4, 1.0 - 1e-4)

    neg = jnp.float32(-1.0)
    pad_w = jnp.full((G * C, H, 1), neg, jnp.float32)
    left = jnp.concatenate([pad_w, heat[:, :, : W - 1]], axis=2)
    right = jnp.concatenate([heat[:, :, 1:], pad_w], axis=2)
    hw = jnp.maximum(jnp.maximum(left, right), heat)
    pad_h = jnp.full((G * C, 1, W), neg, jnp.float32)
    up = jnp.concatenate([pad_h, hw[:, : H - 1, :]], axis=1)
    down = jnp.concatenate([hw[:, 1:, :], pad_h], axis=1)
    hmax = jnp.maximum(jnp.maximum(up, down), hw)

    sup = jnp.where(heat == hmax, heat, 0.0)
    m2 = jnp.max(sup, axis=(1, 2)).reshape(G, C)
    pad_m2 = jnp.full((1, 128 - C), neg, jnp.float32)
    for n in range(G):
        s_refs[n][...] = sup[n * C:(n + 1) * C]
        m2_refs[n][...] = jnp.concatenate([m2[n:n + 1], pad_m2], axis=1)
        q_refs[n][...] = jnp.zeros_like(q_refs[n])

    lane = lax.broadcasted_iota(jnp.int32, (1, 128), 1)
    flat2 = (lax.broadcasted_iota(jnp.int32, (H, W), 0) * W
             + lax.broadcasted_iota(jnp.int32, (H, W), 1))

    def body(k, carry):
        m2s, mvs, cvs = [], [], []
        for n in range(G):
            m2n = m2_refs[n][...]
            mv = jnp.max(m2n, axis=(0, 1), keepdims=True)
            cv = jnp.min(jnp.where(m2n == mv, lane, _BIG), axis=(0, 1),
                         keepdims=True)
            m2s.append(m2n)
            mvs.append(mv)
            cvs.append(cv)
        cs, slabs, posvs = [], [], []
        for n in range(G):
            c = cvs[n][0, 0]
            slab = s_refs[n][c]
            posv = jnp.min(jnp.where(slab == mvs[n], flat2, _BIG),
                           axis=(0, 1), keepdims=True)
            cs.append(c)
            slabs.append(slab)
            posvs.append(posv)
        for n in range(G):
            new_slab = jnp.where(flat2 == posvs[n], neg, slabs[n])
            s_refs[n][cs[n]] = new_slab
            cmaxv = jnp.max(new_slab, axis=(0, 1), keepdims=True)
            m2_refs[n][...] = jnp.where(lane == cvs[n], cmaxv, m2s[n])

            vec3 = jnp.concatenate([
                mvs[n], cvs[n].astype(jnp.float32),
                posvs[n].astype(jnp.float32),
            ], axis=0)
            q3 = q_refs[n][pl.ds(0, 3), :]
            q_refs[n][pl.ds(0, 3), :] = jnp.where(lane == k, vec3, q3)
        return carry

    lax.fori_loop(0, K, body, 0, unroll=False)

    for n in range(G):
        q = q_refs[n][...]
        score = q[0:1, :K]
        clsv = q[1:2, :K]
        posi = q[2:3, :].astype(jnp.int32)
        yi = posi // W
        xi = posi % W

        onehot_y = (lax.broadcasted_iota(jnp.int32, (H, 128), 0)
                    == jnp.broadcast_to(yi, (H, 128))).astype(jnp.float32)
        onehot_x = (lax.broadcasted_iota(jnp.int32, (W, 128), 0)
                    == jnp.broadcast_to(xi, (W, 128))).astype(jnp.float32)

        def gather2(arr):
            t = lax.dot_general(
                arr, onehot_y, (((0,), (0,)), ((), ())),
                precision=lax.Precision.HIGHEST,
                preferred_element_type=jnp.float32)
            return jnp.sum(t * onehot_x, axis=0, keepdims=True)

        g_reg0 = gather2(reg_ref[n, 0])
        g_reg1 = gather2(reg_ref[n, 1])
        g_wh0 = gather2(wh_ref[n, 0])
        g_wh1 = gather2(wh_ref[n, 1])

        ys = yi.astype(jnp.float32)[:, :K] + g_reg1[:, :K]
        xs = xi.astype(jnp.float32)[:, :K] + g_reg0[:, :K]
        wv = g_wh0[:, :K]
        hv = g_wh1[:, :K]
        x1 = (xs - wv * 0.5) * _DOWN_RATIO
        y1 = (ys - hv * 0.5) * _DOWN_RATIO
        x2 = (xs + wv * 0.5) * _DOWN_RATIO
        y2 = (ys + hv * 0.5) * _DOWN_RATIO
        boxes_ref[n] = jnp.concatenate([x1, y1, x2, y2], axis=0)
        scores_ref[n] = score
        cls_ref[n] = clsv


def kernel(hm, wh, reg):
    B, C, H, W = hm.shape
    K = _K
    G = _G
    body = functools.partial(_decode_body, C=C, H=H, W=W, K=K, G=G)
    boxes_t, scores, classes = pl.pallas_call(
        body,
        grid=(B // G,),
        in_specs=[
            pl.BlockSpec((G, C, H, W), lambda b: (b, 0, 0, 0)),
            pl.BlockSpec((G, 2, H, W), lambda b: (b, 0, 0, 0)),
            pl.BlockSpec((G, 2, H, W), lambda b: (b, 0, 0, 0)),
        ],
        out_specs=[
            pl.BlockSpec((G, 4, K), lambda b: (b, 0, 0)),
            pl.BlockSpec((G, 1, K), lambda b: (b, 0, 0)),
            pl.BlockSpec((G, 1, K), lambda b: (b, 0, 0)),
        ],
        out_shape=[
            jax.ShapeDtypeStruct((B, 4, K), jnp.float32),
            jax.ShapeDtypeStruct((B, 1, K), jnp.float32),
            jax.ShapeDtypeStruct((B, 1, K), jnp.float32),
        ],
        scratch_shapes=(
            [pltpu.VMEM((C, H, W), jnp.float32) for _ in range(G)]
            + [pltpu.VMEM((1, 128), jnp.float32) for _ in range(G)]
            + [pltpu.VMEM((8, 128), jnp.float32) for _ in range(G)]
        ),
    )(hm, wh, reg)
    boxes = jnp.transpose(boxes_t, (0, 2, 1))
    return boxes, scores[:, 0, :], classes[:, 0, :]


# speedup vs baseline: 3.0064x; 1.3034x over previous
import functools

import jax
import jax.numpy as jnp
from jax import lax
from jax.experimental import pallas as pl
from jax.experimental.pallas import tpu as pltpu

_DOWN_RATIO = 4.0
_K = 100
_BIG = 2**30
_G = 4


def _decode_body(hm_ref, wh_ref, reg_ref, boxes_ref, scores_ref, cls_ref,
                 *refs, C, H, W, K, G):
    m2_refs = refs[0:G]
    q_refs = refs[G:2 * G]
    neg = jnp.float32(-1.0)
    pad_w = jnp.full((C, H, 1), neg, jnp.float32)
    pad_h = jnp.full((C, 1, W), neg, jnp.float32)
    pad_m2 = jnp.full((1, 128 - C), neg, jnp.float32)
    for n in range(G):
        h = hm_ref[n]
        heat = jnp.clip(jax.nn.sigmoid(h), 1e-4, 1.0 - 1e-4)
        left = jnp.concatenate([pad_w, heat[:, :, : W - 1]], axis=2)
        right = jnp.concatenate([heat[:, :, 1:], pad_w], axis=2)
        hw = jnp.maximum(jnp.maximum(left, right), heat)
        up = jnp.concatenate([pad_h, hw[:, : H - 1, :]], axis=1)
        down = jnp.concatenate([hw[:, 1:, :], pad_h], axis=1)
        hmax = jnp.maximum(jnp.maximum(up, down), hw)
        sup = jnp.where(heat == hmax, heat, 0.0)
        hm_ref[n] = sup
        m2 = jnp.max(sup, axis=(1, 2))[None]
        m2_refs[n][...] = jnp.concatenate([m2, pad_m2], axis=1)
        q_refs[n][...] = jnp.zeros_like(q_refs[n])

    lane = lax.broadcasted_iota(jnp.int32, (1, 128), 1)
    flat2 = (lax.broadcasted_iota(jnp.int32, (H, W), 0) * W
             + lax.broadcasted_iota(jnp.int32, (H, W), 1))

    def body(k, carry):
        m2s, mvs, cvs = [], [], []
        for n in range(G):
            m2n = m2_refs[n][...]
            mv = jnp.max(m2n, axis=(0, 1), keepdims=True)
            cv = jnp.min(jnp.where(m2n == mv, lane, _BIG), axis=(0, 1),
                         keepdims=True)
            m2s.append(m2n)
            mvs.append(mv)
            cvs.append(cv)
        cs, slabs, posvs = [], [], []
        for n in range(G):
            c = cvs[n][0, 0]
            slab = hm_ref[n, c]
            posv = jnp.min(jnp.where(slab == mvs[n], flat2, _BIG),
                           axis=(0, 1), keepdims=True)
            cs.append(c)
            slabs.append(slab)
            posvs.append(posv)
        for n in range(G):
            new_slab = jnp.where(flat2 == posvs[n], neg, slabs[n])
            hm_ref[n, cs[n]] = new_slab
            cmaxv = jnp.max(new_slab, axis=(0, 1), keepdims=True)
            m2_refs[n][...] = jnp.where(lane == cvs[n], cmaxv, m2s[n])

            vec3 = jnp.concatenate([
                mvs[n], cvs[n].astype(jnp.float32),
                posvs[n].astype(jnp.float32),
            ], axis=0)
            q3 = q_refs[n][pl.ds(0, 3), :]
            q_refs[n][pl.ds(0, 3), :] = jnp.where(lane == k, vec3, q3)
        return carry

    lax.fori_loop(0, K, body, 0, unroll=False)

    for n in range(G):
        q = q_refs[n][...]
        score = q[0:1, :K]
        clsv = q[1:2, :K]
        posi = q[2:3, :].astype(jnp.int32)
        yi = posi // W
        xi = posi % W

        onehot_y = (lax.broadcasted_iota(jnp.int32, (H, 128), 0)
                    == jnp.broadcast_to(yi, (H, 128))).astype(jnp.float32)
        onehot_x = (lax.broadcasted_iota(jnp.int32, (W, 128), 0)
                    == jnp.broadcast_to(xi, (W, 128))).astype(jnp.float32)

        def gather2(arr):
            t = lax.dot_general(
                arr, onehot_y, (((0,), (0,)), ((), ())),
                precision=lax.Precision.HIGHEST,
                preferred_element_type=jnp.float32)
            return jnp.sum(t * onehot_x, axis=0, keepdims=True)

        g_reg0 = gather2(reg_ref[n, 0])
        g_reg1 = gather2(reg_ref[n, 1])
        g_wh0 = gather2(wh_ref[n, 0])
        g_wh1 = gather2(wh_ref[n, 1])

        ys = yi.astype(jnp.float32)[:, :K] + g_reg1[:, :K]
        xs = xi.astype(jnp.float32)[:, :K] + g_reg0[:, :K]
        wv = g_wh0[:, :K]
        hv = g_wh1[:, :K]
        x1 = (xs - wv * 0.5) * _DOWN_RATIO
        y1 = (ys - hv * 0.5) * _DOWN_RATIO
        x2 = (xs + wv * 0.5) * _DOWN_RATIO
        y2 = (ys + hv * 0.5) * _DOWN_RATIO
        boxes_ref[n] = jnp.concatenate([x1, y1, x2, y2], axis=0)
        scores_ref[n] = score
        cls_ref[n] = clsv


def kernel(hm, wh, reg):
    B, C, H, W = hm.shape
    K = _K
    G = _G
    body = functools.partial(_decode_body, C=C, H=H, W=W, K=K, G=G)
    boxes_t, scores, classes = pl.pallas_call(
        body,
        grid=(B // G,),
        in_specs=[
            pl.BlockSpec((G, C, H, W), lambda b: (b, 0, 0, 0)),
            pl.BlockSpec((G, 2, H, W), lambda b: (b, 0, 0, 0)),
            pl.BlockSpec((G, 2, H, W), lambda b: (b, 0, 0, 0)),
        ],
        out_specs=[
            pl.BlockSpec((G, 4, K), lambda b: (b, 0, 0)),
            pl.BlockSpec((G, 1, K), lambda b: (b, 0, 0)),
            pl.BlockSpec((G, 1, K), lambda b: (b, 0, 0)),
        ],
        out_shape=[
            jax.ShapeDtypeStruct((B, 4, K), jnp.float32),
            jax.ShapeDtypeStruct((B, 1, K), jnp.float32),
            jax.ShapeDtypeStruct((B, 1, K), jnp.float32),
        ],
        scratch_shapes=(
            [pltpu.VMEM((1, 128), jnp.float32) for _ in range(G)]
            + [pltpu.VMEM((8, 128), jnp.float32) for _ in range(G)]
        ),
    )(hm, wh, reg)
    boxes = jnp.transpose(boxes_t, (0, 2, 1))
    return boxes, scores[:, 0, :], classes[:, 0, :]


# speedup vs baseline: 5.6271x; 1.8717x over previous
import functools

import jax
import jax.numpy as jnp
from jax import lax
from jax.experimental import pallas as pl
from jax.experimental.pallas import tpu as pltpu

_DOWN_RATIO = 4.0
_K = 100
_BIG = 2**30
_G = 4


def _decode_body(hm_ref, wh_ref, reg_ref, boxes_ref, scores_ref, cls_ref,
                 *refs, C, H, W, K, G):
    m2_ref, q_ref = refs
    neg = jnp.float32(-1.0)
    pad_w = jnp.full((C, H, 1), neg, jnp.float32)
    pad_h = jnp.full((C, 1, W), neg, jnp.float32)
    pad_m2 = jnp.full((1, 128 - C), neg, jnp.float32)
    for n in range(G):
        h = hm_ref[n]
        heat = jnp.clip(jax.nn.sigmoid(h), 1e-4, 1.0 - 1e-4)
        left = jnp.concatenate([pad_w, heat[:, :, : W - 1]], axis=2)
        right = jnp.concatenate([heat[:, :, 1:], pad_w], axis=2)
        hw = jnp.maximum(jnp.maximum(left, right), heat)
        up = jnp.concatenate([pad_h, hw[:, : H - 1, :]], axis=1)
        down = jnp.concatenate([hw[:, 1:, :], pad_h], axis=1)
        hmax = jnp.maximum(jnp.maximum(up, down), hw)
        sup = jnp.where(heat == hmax, heat, 0.0)
        hm_ref[n] = sup
        m2 = jnp.max(sup, axis=(1, 2))[None]
        m2_ref[pl.ds(n, 1), :] = jnp.concatenate([m2, pad_m2], axis=1)
    q_ref[...] = jnp.zeros_like(q_ref)

    laneG = lax.broadcasted_iota(jnp.int32, (G, 128), 1)
    flat2 = (lax.broadcasted_iota(jnp.int32, (H, W), 0) * W
             + lax.broadcasted_iota(jnp.int32, (H, W), 1))

    def body(k, carry):
        m2 = m2_ref[...]
        mvs = jnp.max(m2, axis=1, keepdims=True)
        cvs = jnp.min(jnp.where(m2 == mvs, laneG, _BIG), axis=1,
                      keepdims=True)
        cs, slabs, midxs = [], [], []
        for n in range(G):
            c = cvs[n, 0]
            slab = hm_ref[n, c]
            midx = jnp.min(jnp.where(slab == mvs[n:n + 1], flat2, _BIG),
                           axis=0, keepdims=True)
            cs.append(c)
            slabs.append(slab)
            midxs.append(midx)
        midx_all = jnp.concatenate(midxs, axis=0)
        posvs = jnp.min(midx_all, axis=1, keepdims=True)
        lmaxs = []
        for n in range(G):
            new_slab = jnp.where(flat2 == posvs[n:n + 1], neg, slabs[n])
            hm_ref[n, cs[n]] = new_slab
            lmaxs.append(jnp.max(new_slab, axis=0, keepdims=True))
        lmax_all = jnp.concatenate(lmaxs, axis=0)
        cmaxs = jnp.max(lmax_all, axis=1, keepdims=True)
        m2_ref[...] = jnp.where(laneG == cvs, cmaxs, m2)

        sel = laneG == k
        q_ref[0] = jnp.where(sel, mvs, q_ref[0])
        q_ref[1] = jnp.where(sel, cvs.astype(jnp.float32), q_ref[1])
        q_ref[2] = jnp.where(sel, posvs.astype(jnp.float32), q_ref[2])
        return carry

    lax.fori_loop(0, K, body, 0, unroll=False)

    for n in range(G):
        score = q_ref[0, n:n + 1, :K]
        clsv = q_ref[1, n:n + 1, :K]
        posi = q_ref[2, n:n + 1, :].astype(jnp.int32)
        yi = posi // W
        xi = posi % W

        onehot_y = (lax.broadcasted_iota(jnp.int32, (H, 128), 0)
                    == jnp.broadcast_to(yi, (H, 128))).astype(jnp.float32)
        onehot_x = (lax.broadcasted_iota(jnp.int32, (W, 128), 0)
                    == jnp.broadcast_to(xi, (W, 128))).astype(jnp.float32)

        def gather2(arr):
            t = lax.dot_general(
                arr, onehot_y, (((0,), (0,)), ((), ())),
                precision=lax.Precision.HIGHEST,
                preferred_element_type=jnp.float32)
            return jnp.sum(t * onehot_x, axis=0, keepdims=True)

        g_reg0 = gather2(reg_ref[n, 0])
        g_reg1 = gather2(reg_ref[n, 1])
        g_wh0 = gather2(wh_ref[n, 0])
        g_wh1 = gather2(wh_ref[n, 1])

        ys = yi.astype(jnp.float32)[:, :K] + g_reg1[:, :K]
        xs = xi.astype(jnp.float32)[:, :K] + g_reg0[:, :K]
        wv = g_wh0[:, :K]
        hv = g_wh1[:, :K]
        x1 = (xs - wv * 0.5) * _DOWN_RATIO
        y1 = (ys - hv * 0.5) * _DOWN_RATIO
        x2 = (xs + wv * 0.5) * _DOWN_RATIO
        y2 = (ys + hv * 0.5) * _DOWN_RATIO
        boxes_ref[n] = jnp.concatenate([x1, y1, x2, y2], axis=0)
        scores_ref[n] = score
        cls_ref[n] = clsv


def kernel(hm, wh, reg):
    B, C, H, W = hm.shape
    K = _K
    G = _G
    body = functools.partial(_decode_body, C=C, H=H, W=W, K=K, G=G)
    boxes_t, scores, classes = pl.pallas_call(
        body,
        grid=(B // G,),
        in_specs=[
            pl.BlockSpec((G, C, H, W), lambda b: (b, 0, 0, 0)),
            pl.BlockSpec((G, 2, H, W), lambda b: (b, 0, 0, 0)),
            pl.BlockSpec((G, 2, H, W), lambda b: (b, 0, 0, 0)),
        ],
        out_specs=[
            pl.BlockSpec((G, 4, K), lambda b: (b, 0, 0)),
            pl.BlockSpec((G, 1, K), lambda b: (b, 0, 0)),
            pl.BlockSpec((G, 1, K), lambda b: (b, 0, 0)),
        ],
        out_shape=[
            jax.ShapeDtypeStruct((B, 4, K), jnp.float32),
            jax.ShapeDtypeStruct((B, 1, K), jnp.float32),
            jax.ShapeDtypeStruct((B, 1, K), jnp.float32),
        ],
        scratch_shapes=[
            pltpu.VMEM((G, 128), jnp.float32),
            pltpu.VMEM((3, G, 128), jnp.float32),
        ],
    )(hm, wh, reg)
    boxes = jnp.transpose(boxes_t, (0, 2, 1))
    return boxes, scores[:, 0, :], classes[:, 0, :]


# speedup vs baseline: 8.0159x; 1.4245x over previous
import functools

import jax
import jax.numpy as jnp
from jax import lax
from jax.experimental import pallas as pl
from jax.experimental.pallas import tpu as pltpu

_DOWN_RATIO = 4.0
_K = 100
_BIG = 2**30
_G = 4


def _decode_body(hm_ref, wh_ref, reg_ref, boxes_ref, scores_ref, cls_ref,
                 *refs, C, H, W, K, G):
    m2_ref, q_ref = refs
    neg = jnp.float32(-1.0)
    pad_w = jnp.full((C, H, 1), neg, jnp.float32)
    pad_h = jnp.full((C, 1, W), neg, jnp.float32)
    pad_m2 = jnp.full((1, 128 - C), neg, jnp.float32)
    for n in range(G):
        h = hm_ref[n]
        heat = jnp.clip(jax.nn.sigmoid(h), 1e-4, 1.0 - 1e-4)
        left = jnp.concatenate([pad_w, heat[:, :, : W - 1]], axis=2)
        right = jnp.concatenate([heat[:, :, 1:], pad_w], axis=2)
        hw = jnp.maximum(jnp.maximum(left, right), heat)
        up = jnp.concatenate([pad_h, hw[:, : H - 1, :]], axis=1)
        down = jnp.concatenate([hw[:, 1:, :], pad_h], axis=1)
        hmax = jnp.maximum(jnp.maximum(up, down), hw)
        sup = jnp.where(heat == hmax, heat, 0.0)
        hm_ref[n] = sup
        m2 = jnp.max(sup, axis=(1, 2))[None]
        m2_ref[pl.ds(n, 1), :] = jnp.concatenate([m2, pad_m2], axis=1)
    q_ref[...] = jnp.zeros_like(q_ref)

    laneG = lax.broadcasted_iota(jnp.int32, (G, 128), 1)
    flat2 = (lax.broadcasted_iota(jnp.int32, (H, W), 0) * W
             + lax.broadcasted_iota(jnp.int32, (H, W), 1))

    m2_0 = m2_ref[...]
    mvs0 = jnp.max(m2_0, axis=1, keepdims=True)
    cvs0 = jnp.min(jnp.where(m2_0 == mvs0, laneG, _BIG), axis=1,
                   keepdims=True)

    def body(k, carry):
        mvs, cvs = carry
        cs, slabs, midxs = [], [], []
        for n in range(G):
            c = cvs[n, 0]
            slab = hm_ref[n, c]
            midx = jnp.min(jnp.where(slab == mvs[n:n + 1], flat2, _BIG),
                           axis=0, keepdims=True)
            cs.append(c)
            slabs.append(slab)
            midxs.append(midx)
        midx_all = jnp.concatenate(midxs, axis=0)
        posvs = jnp.min(midx_all, axis=1, keepdims=True)

        m2 = m2_ref[...]
        m2m = jnp.where(laneG == cvs, jnp.float32(-2.0), m2)
        sec_v = jnp.max(m2m, axis=1, keepdims=True)
        sec_c = jnp.min(jnp.where(m2m == sec_v, laneG, _BIG), axis=1,
                        keepdims=True)

        lmaxs = []
        for n in range(G):
            new_slab = jnp.where(flat2 == posvs[n:n + 1], neg, slabs[n])
            hm_ref[n, cs[n]] = new_slab
            lmaxs.append(jnp.max(new_slab, axis=0, keepdims=True))
        lmax_all = jnp.concatenate(lmaxs, axis=0)
        cmaxs = jnp.max(lmax_all, axis=1, keepdims=True)
        m2_ref[...] = jnp.where(laneG == cvs, cmaxs, m2)

        take_c = (cmaxs > sec_v) | ((cmaxs == sec_v) & (cvs <= sec_c))
        nmvs = jnp.where(take_c, cmaxs, sec_v)
        ncvs = jnp.where(take_c, cvs, sec_c)

        sel = laneG == k
        q_ref[0] = jnp.where(sel, mvs, q_ref[0])
        q_ref[1] = jnp.where(sel, cvs.astype(jnp.float32), q_ref[1])
        q_ref[2] = jnp.where(sel, posvs.astype(jnp.float32), q_ref[2])
        return (nmvs, ncvs)

    lax.fori_loop(0, K, body, (mvs0, cvs0), unroll=False)

    for n in range(G):
        score = q_ref[0, n:n + 1, :K]
        clsv = q_ref[1, n:n + 1, :K]
        posi = q_ref[2, n:n + 1, :].astype(jnp.int32)
        yi = posi // W
        xi = posi % W

        onehot_y = (lax.broadcasted_iota(jnp.int32, (H, 128), 0)
                    == jnp.broadcast_to(yi, (H, 128))).astype(jnp.float32)
        onehot_x = (lax.broadcasted_iota(jnp.int32, (W, 128), 0)
                    == jnp.broadcast_to(xi, (W, 128))).astype(jnp.float32)

        def gather2(arr):
            t = lax.dot_general(
                arr, onehot_y, (((0,), (0,)), ((), ())),
                precision=lax.Precision.HIGHEST,
                preferred_element_type=jnp.float32)
            return jnp.sum(t * onehot_x, axis=0, keepdims=True)

        g_reg0 = gather2(reg_ref[n, 0])
        g_reg1 = gather2(reg_ref[n, 1])
        g_wh0 = gather2(wh_ref[n, 0])
        g_wh1 = gather2(wh_ref[n, 1])

        ys = yi.astype(jnp.float32)[:, :K] + g_reg1[:, :K]
        xs = xi.astype(jnp.float32)[:, :K] + g_reg0[:, :K]
        wv = g_wh0[:, :K]
        hv = g_wh1[:, :K]
        x1 = (xs - wv * 0.5) * _DOWN_RATIO
        y1 = (ys - hv * 0.5) * _DOWN_RATIO
        x2 = (xs + wv * 0.5) * _DOWN_RATIO
        y2 = (ys + hv * 0.5) * _DOWN_RATIO
        boxes_ref[n] = jnp.concatenate([x1, y1, x2, y2], axis=0)
        scores_ref[n] = score
        cls_ref[n] = clsv


def kernel(hm, wh, reg):
    B, C, H, W = hm.shape
    K = _K
    G = _G
    body = functools.partial(_decode_body, C=C, H=H, W=W, K=K, G=G)
    boxes_t, scores, classes = pl.pallas_call(
        body,
        grid=(B // G,),
        in_specs=[
            pl.BlockSpec((G, C, H, W), lambda b: (b, 0, 0, 0)),
            pl.BlockSpec((G, 2, H, W), lambda b: (b, 0, 0, 0)),
            pl.BlockSpec((G, 2, H, W), lambda b: (b, 0, 0, 0)),
        ],
        out_specs=[
            pl.BlockSpec((G, 4, K), lambda b: (b, 0, 0)),
            pl.BlockSpec((G, 1, K), lambda b: (b, 0, 0)),
            pl.BlockSpec((G, 1, K), lambda b: (b, 0, 0)),
        ],
        out_shape=[
            jax.ShapeDtypeStruct((B, 4, K), jnp.float32),
            jax.ShapeDtypeStruct((B, 1, K), jnp.float32),
            jax.ShapeDtypeStruct((B, 1, K), jnp.float32),
        ],
        scratch_shapes=[
            pltpu.VMEM((G, 128), jnp.float32),
            pltpu.VMEM((3, G, 128), jnp.float32),
        ],
    )(hm, wh, reg)
    boxes = jnp.transpose(boxes_t, (0, 2, 1))
    return boxes, scores[:, 0, :], classes[:, 0, :]


# speedup vs baseline: 8.9158x; 1.1123x over previous
import functools

import jax
import jax.numpy as jnp
from jax import lax
from jax.experimental import pallas as pl
from jax.experimental.pallas import tpu as pltpu

_DOWN_RATIO = 4.0
_K = 100
_BIG = 2**30
_G = 4


def _decode_body(hm_ref, wh_ref, reg_ref, boxes_ref, scores_ref, cls_ref,
                 *refs, C, H, W, K, G):
    m2_ref, q_ref = refs
    neg = jnp.float32(-1.0)
    pad_w = jnp.full((C, H, 1), neg, jnp.float32)
    pad_h = jnp.full((C, 1, W), neg, jnp.float32)
    pad_m2 = jnp.full((1, 128 - C), neg, jnp.float32)
    for n in range(G):
        h = hm_ref[n]
        heat = jnp.clip(jax.nn.sigmoid(h), 1e-4, 1.0 - 1e-4)
        left = jnp.concatenate([pad_w, heat[:, :, : W - 1]], axis=2)
        right = jnp.concatenate([heat[:, :, 1:], pad_w], axis=2)
        hw = jnp.maximum(jnp.maximum(left, right), heat)
        up = jnp.concatenate([pad_h, hw[:, : H - 1, :]], axis=1)
        down = jnp.concatenate([hw[:, 1:, :], pad_h], axis=1)
        hmax = jnp.maximum(jnp.maximum(up, down), hw)
        sup = jnp.where(heat == hmax, heat, 0.0)
        hm_ref[n] = sup
        m2 = jnp.max(sup, axis=(1, 2))[None]
        m2_ref[pl.ds(n, 1), :] = jnp.concatenate([m2, pad_m2], axis=1)
    q_ref[...] = jnp.zeros_like(q_ref)

    laneG = lax.broadcasted_iota(jnp.int32, (G, 128), 1)
    flat2 = (lax.broadcasted_iota(jnp.int32, (H, W), 0) * W
             + lax.broadcasted_iota(jnp.int32, (H, W), 1))

    m2_0 = m2_ref[...]
    mvs0 = jnp.max(m2_0, axis=1, keepdims=True)
    cvs0 = jnp.min(jnp.where(m2_0 == mvs0, laneG, _BIG), axis=1,
                   keepdims=True)

    def body(k, carry):
        mvs, cvs = carry
        cs, slabs, midxs, avecs, cntvecs = [], [], [], [], []
        for n in range(G):
            c = cvs[n, 0]
            slab = hm_ref[n, c]
            h2 = H // 2
            cond = slab[:h2] >= slab[h2:]
            v1 = jnp.where(cond, slab[:h2], slab[h2:])
            i1 = jnp.where(cond, flat2[:h2], flat2[h2:])
            v2 = jnp.minimum(slab[:h2], slab[h2:])
            h = h2
            while h > 1:
                h2 = h // 2
                cond = ((v1[:h2] > v1[h2:])
                        | ((v1[:h2] == v1[h2:]) & (i1[:h2] <= i1[h2:])))
                nv2 = jnp.maximum(jnp.minimum(v1[:h2], v1[h2:]),
                                  jnp.where(cond, v2[:h2], v2[h2:]))
                i1 = jnp.where(cond, i1[:h2], i1[h2:])
                v1 = jnp.where(cond, v1[:h2], v1[h2:])
                v2 = nv2
                h = h2
            is_max = v1 == mvs[n:n + 1]
            midxs.append(jnp.where(is_max, i1, _BIG))
            avecs.append(jnp.where(is_max, v2, v1))
            cntvecs.append(is_max.astype(jnp.int32))
            cs.append(c)
            slabs.append(slab)
        posvs = jnp.min(jnp.concatenate(midxs, axis=0), axis=1,
                        keepdims=True)
        avs = jnp.max(jnp.concatenate(avecs, axis=0), axis=1,
                      keepdims=True)
        cnts = jnp.sum(jnp.concatenate(cntvecs, axis=0), axis=1,
                       keepdims=True)
        cmaxs = jnp.where(cnts >= 2, mvs, avs)

        m2 = m2_ref[...]
        m2m = jnp.where(laneG == cvs, jnp.float32(-2.0), m2)
        sec_v = jnp.max(m2m, axis=1, keepdims=True)
        sec_c = jnp.min(jnp.where(m2m == sec_v, laneG, _BIG), axis=1,
                        keepdims=True)

        for n in range(G):
            hm_ref[n, cs[n]] = jnp.where(flat2 == posvs[n:n + 1], neg,
                                         slabs[n])
        m2_ref[...] = jnp.where(laneG == cvs, cmaxs, m2)

        take_c = (cmaxs > sec_v) | ((cmaxs == sec_v) & (cvs <= sec_c))
        nmvs = jnp.where(take_c, cmaxs, sec_v)
        ncvs = jnp.where(take_c, cvs, sec_c)

        sel = laneG == k
        q_ref[0] = jnp.where(sel, mvs, q_ref[0])
        q_ref[1] = jnp.where(sel, cvs.astype(jnp.float32), q_ref[1])
        q_ref[2] = jnp.where(sel, posvs.astype(jnp.float32), q_ref[2])
        return (nmvs, ncvs)

    lax.fori_loop(0, K, body, (mvs0, cvs0), unroll=False)

    for n in range(G):
        score = q_ref[0, n:n + 1, :K]
        clsv = q_ref[1, n:n + 1, :K]
        posi = q_ref[2, n:n + 1, :].astype(jnp.int32)
        yi = posi // W
        xi = posi % W

        onehot_y = (lax.broadcasted_iota(jnp.int32, (H, 128), 0)
                    == jnp.broadcast_to(yi, (H, 128))).astype(jnp.float32)
        onehot_x = (lax.broadcasted_iota(jnp.int32, (W, 128), 0)
                    == jnp.broadcast_to(xi, (W, 128))).astype(jnp.float32)

        def gather2(arr):
            t = lax.dot_general(
                arr, onehot_y, (((0,), (0,)), ((), ())),
                precision=lax.Precision.HIGHEST,
                preferred_element_type=jnp.float32)
            return jnp.sum(t * onehot_x, axis=0, keepdims=True)

        g_reg0 = gather2(reg_ref[n, 0])
        g_reg1 = gather2(reg_ref[n, 1])
        g_wh0 = gather2(wh_ref[n, 0])
        g_wh1 = gather2(wh_ref[n, 1])

        ys = yi.astype(jnp.float32)[:, :K] + g_reg1[:, :K]
        xs = xi.astype(jnp.float32)[:, :K] + g_reg0[:, :K]
        wv = g_wh0[:, :K]
        hv = g_wh1[:, :K]
        x1 = (xs - wv * 0.5) * _DOWN_RATIO
        y1 = (ys - hv * 0.5) * _DOWN_RATIO
        x2 = (xs + wv * 0.5) * _DOWN_RATIO
        y2 = (ys + hv * 0.5) * _DOWN_RATIO
        boxes_ref[n] = jnp.concatenate([x1, y1, x2, y2], axis=0)
        scores_ref[n] = score
        cls_ref[n] = clsv


def kernel(hm, wh, reg):
    B, C, H, W = hm.shape
    K = _K
    G = _G
    body = functools.partial(_decode_body, C=C, H=H, W=W, K=K, G=G)
    boxes_t, scores, classes = pl.pallas_call(
        body,
        grid=(B // G,),
        in_specs=[
            pl.BlockSpec((G, C, H, W), lambda b: (b, 0, 0, 0)),
            pl.BlockSpec((G, 2, H, W), lambda b: (b, 0, 0, 0)),
            pl.BlockSpec((G, 2, H, W), lambda b: (b, 0, 0, 0)),
        ],
        out_specs=[
            pl.BlockSpec((G, 4, K), lambda b: (b, 0, 0)),
            pl.BlockSpec((G, 1, K), lambda b: (b, 0, 0)),
            pl.BlockSpec((G, 1, K), lambda b: (b, 0, 0)),
        ],
        out_shape=[
            jax.ShapeDtypeStruct((B, 4, K), jnp.float32),
            jax.ShapeDtypeStruct((B, 1, K), jnp.float32),
            jax.ShapeDtypeStruct((B, 1, K), jnp.float32),
        ],
        scratch_shapes=[
            pltpu.VMEM((G, 128), jnp.float32),
            pltpu.VMEM((3, G, 128), jnp.float32),
        ],
    )(hm, wh, reg)
    boxes = jnp.transpose(boxes_t, (0, 2, 1))
    return boxes, scores[:, 0, :], classes[:, 0, :]


# speedup vs baseline: 9.2239x; 1.0345x over previous
import functools

import jax
import jax.numpy as jnp
from jax import lax
from jax.experimental import pallas as pl
from jax.experimental.pallas import tpu as pltpu

_DOWN_RATIO = 4.0
_K = 100
_BIG = 2**30
_G = 4


def _decode_body(hm_ref, wh_ref, reg_ref, boxes_ref, scores_ref, cls_ref,
                 *refs, C, H, W, K, G):
    m2_ref, q_ref = refs
    neg = jnp.float32(-1.0)
    pad_w = jnp.full((C, H, 1), neg, jnp.float32)
    pad_h = jnp.full((C, 1, W), neg, jnp.float32)
    pad_m2 = jnp.full((1, 128 - C), neg, jnp.float32)
    for n in range(G):
        h = hm_ref[n]
        heat = jnp.clip(jax.nn.sigmoid(h), 1e-4, 1.0 - 1e-4)
        left = jnp.concatenate([pad_w, heat[:, :, : W - 1]], axis=2)
        right = jnp.concatenate([heat[:, :, 1:], pad_w], axis=2)
        hw = jnp.maximum(jnp.maximum(left, right), heat)
        up = jnp.concatenate([pad_h, hw[:, : H - 1, :]], axis=1)
        down = jnp.concatenate([hw[:, 1:, :], pad_h], axis=1)
        hmax = jnp.maximum(jnp.maximum(up, down), hw)
        sup = jnp.where(heat == hmax, heat, 0.0)
        hm_ref[n] = sup
        m2 = jnp.max(sup, axis=(1, 2))[None]
        m2_ref[pl.ds(n, 1), :] = jnp.concatenate([m2, pad_m2], axis=1)
    q_ref[...] = jnp.zeros_like(q_ref)

    laneG = lax.broadcasted_iota(jnp.int32, (G, 128), 1)
    flat2 = (lax.broadcasted_iota(jnp.int32, (H, W), 0) * W
             + lax.broadcasted_iota(jnp.int32, (H, W), 1))

    m2_0 = m2_ref[...]
    mvs0 = jnp.max(m2_0, axis=1, keepdims=True)
    cvs0 = jnp.min(jnp.where(m2_0 == mvs0, laneG, _BIG), axis=1,
                   keepdims=True)

    def body(k, carry):
        mvs, cvs = carry
        cs, slabs, midxs, avecs, cntvecs = [], [], [], [], []
        for n in range(G):
            c = cvs[n, 0]
            slab = hm_ref[n, c]
            h2 = H // 2
            cond = slab[:h2] >= slab[h2:]
            v1 = jnp.where(cond, slab[:h2], slab[h2:])
            i1 = jnp.where(cond, flat2[:h2], flat2[h2:])
            v2 = jnp.minimum(slab[:h2], slab[h2:])
            h = h2
            while h > 1:
                h2 = h // 2
                cond = ((v1[:h2] > v1[h2:])
                        | ((v1[:h2] == v1[h2:]) & (i1[:h2] <= i1[h2:])))
                nv2 = jnp.maximum(jnp.minimum(v1[:h2], v1[h2:]),
                                  jnp.where(cond, v2[:h2], v2[h2:]))
                i1 = jnp.where(cond, i1[:h2], i1[h2:])
                v1 = jnp.where(cond, v1[:h2], v1[h2:])
                v2 = nv2
                h = h2
            is_max = v1 == mvs[n:n + 1]
            midxs.append(jnp.where(is_max, i1, _BIG))
            avecs.append(jnp.where(is_max, v2, v1))
            cntvecs.append(is_max.astype(jnp.int32))
            cs.append(c)
            slabs.append(slab)
        posvs = jnp.min(jnp.concatenate(midxs, axis=0), axis=1,
                        keepdims=True)
        avs = jnp.max(jnp.concatenate(avecs, axis=0), axis=1,
                      keepdims=True)
        cnts = jnp.sum(jnp.concatenate(cntvecs, axis=0), axis=1,
                       keepdims=True)
        cmaxs = jnp.where(cnts >= 2, mvs, avs)

        m2 = m2_ref[...]
        m2m = jnp.where(laneG == cvs, jnp.float32(-2.0), m2)
        sec_v = jnp.max(m2m, axis=1, keepdims=True)
        sec_c = jnp.min(jnp.where(m2m == sec_v, laneG, _BIG), axis=1,
                        keepdims=True)

        for n in range(G):
            hm_ref[n, cs[n]] = jnp.where(flat2 == posvs[n:n + 1], neg,
                                         slabs[n])
        m2_ref[...] = jnp.where(laneG == cvs, cmaxs, m2)

        take_c = (cmaxs > sec_v) | ((cmaxs == sec_v) & (cvs <= sec_c))
        nmvs = jnp.where(take_c, cmaxs, sec_v)
        ncvs = jnp.where(take_c, cvs, sec_c)

        sel = laneG == k
        q_ref[0] = jnp.where(sel, mvs, q_ref[0])
        q_ref[1] = jnp.where(sel, cvs.astype(jnp.float32), q_ref[1])
        q_ref[2] = jnp.where(sel, posvs.astype(jnp.float32), q_ref[2])
        return (nmvs, ncvs)

    lax.fori_loop(0, K, body, (mvs0, cvs0), unroll=2)

    for n in range(G):
        score = q_ref[0, n:n + 1, :K]
        clsv = q_ref[1, n:n + 1, :K]
        posi = q_ref[2, n:n + 1, :].astype(jnp.int32)
        yi = posi // W
        xi = posi % W

        onehot_y = (lax.broadcasted_iota(jnp.int32, (H, 128), 0)
                    == jnp.broadcast_to(yi, (H, 128))).astype(jnp.float32)
        onehot_x = (lax.broadcasted_iota(jnp.int32, (W, 128), 0)
                    == jnp.broadcast_to(xi, (W, 128))).astype(jnp.float32)

        def gather2(arr):
            t = lax.dot_general(
                arr, onehot_y, (((0,), (0,)), ((), ())),
                precision=lax.Precision.HIGHEST,
                preferred_element_type=jnp.float32)
            return jnp.sum(t * onehot_x, axis=0, keepdims=True)

        g_reg0 = gather2(reg_ref[n, 0])
        g_reg1 = gather2(reg_ref[n, 1])
        g_wh0 = gather2(wh_ref[n, 0])
        g_wh1 = gather2(wh_ref[n, 1])

        ys = yi.astype(jnp.float32)[:, :K] + g_reg1[:, :K]
        xs = xi.astype(jnp.float32)[:, :K] + g_reg0[:, :K]
        wv = g_wh0[:, :K]
        hv = g_wh1[:, :K]
        x1 = (xs - wv * 0.5) * _DOWN_RATIO
        y1 = (ys - hv * 0.5) * _DOWN_RATIO
        x2 = (xs + wv * 0.5) * _DOWN_RATIO
        y2 = (ys + hv * 0.5) * _DOWN_RATIO
        boxes_ref[n] = jnp.concatenate([x1, y1, x2, y2], axis=0)
        scores_ref[n] = score
        cls_ref[n] = clsv


def kernel(hm, wh, reg):
    B, C, H, W = hm.shape
    K = _K
    G = _G
    body = functools.partial(_decode_body, C=C, H=H, W=W, K=K, G=G)
    boxes_t, scores, classes = pl.pallas_call(
        body,
        grid=(B // G,),
        in_specs=[
            pl.BlockSpec((G, C, H, W), lambda b: (b, 0, 0, 0)),
            pl.BlockSpec((G, 2, H, W), lambda b: (b, 0, 0, 0)),
            pl.BlockSpec((G, 2, H, W), lambda b: (b, 0, 0, 0)),
        ],
        out_specs=[
            pl.BlockSpec((G, 4, K), lambda b: (b, 0, 0)),
            pl.BlockSpec((G, 1, K), lambda b: (b, 0, 0)),
            pl.BlockSpec((G, 1, K), lambda b: (b, 0, 0)),
        ],
        out_shape=[
            jax.ShapeDtypeStruct((B, 4, K), jnp.float32),
            jax.ShapeDtypeStruct((B, 1, K), jnp.float32),
            jax.ShapeDtypeStruct((B, 1, K), jnp.float32),
        ],
        scratch_shapes=[
            pltpu.VMEM((G, 128), jnp.float32),
            pltpu.VMEM((3, G, 128), jnp.float32),
        ],
    )(hm, wh, reg)
    boxes = jnp.transpose(boxes_t, (0, 2, 1))
    return boxes, scores[:, 0, :], classes[:, 0, :]


# speedup vs baseline: 9.2678x; 1.0048x over previous
import functools

import jax
import jax.numpy as jnp
from jax import lax
from jax.experimental import pallas as pl
from jax.experimental.pallas import tpu as pltpu

_DOWN_RATIO = 4.0
_K = 100
_BIG = 2**30
_G = 4


def _decode_body(hm_ref, wh_ref, reg_ref, boxes_ref, scores_ref, cls_ref,
                 *refs, C, H, W, K, G):
    m2_ref, q_ref = refs
    neg = jnp.float32(-1.0)
    pad_w = jnp.full((C, H, 1), neg, jnp.float32)
    pad_h = jnp.full((C, 1, W), neg, jnp.float32)
    pad_m2 = jnp.full((1, 128 - C), neg, jnp.float32)
    for n in range(G):
        h = hm_ref[n]
        heat = jnp.clip(jax.nn.sigmoid(h), 1e-4, 1.0 - 1e-4)
        left = jnp.concatenate([pad_w, heat[:, :, : W - 1]], axis=2)
        right = jnp.concatenate([heat[:, :, 1:], pad_w], axis=2)
        hw = jnp.maximum(jnp.maximum(left, right), heat)
        up = jnp.concatenate([pad_h, hw[:, : H - 1, :]], axis=1)
        down = jnp.concatenate([hw[:, 1:, :], pad_h], axis=1)
        hmax = jnp.maximum(jnp.maximum(up, down), hw)
        sup = jnp.where(heat == hmax, heat, 0.0)
        hm_ref[n] = sup
        m2 = jnp.max(sup, axis=(1, 2))[None]
        m2_ref[pl.ds(n, 1), :] = jnp.concatenate([m2, pad_m2], axis=1)
    q_ref[...] = jnp.zeros_like(q_ref)

    laneG = lax.broadcasted_iota(jnp.int32, (G, 128), 1)
    flat2 = (lax.broadcasted_iota(jnp.int32, (H, W), 0) * W
             + lax.broadcasted_iota(jnp.int32, (H, W), 1))

    m2_0 = m2_ref[...]
    mvs0 = jnp.max(m2_0, axis=1, keepdims=True)
    cvs0 = jnp.min(jnp.where(m2_0 == mvs0, laneG, _BIG), axis=1,
                   keepdims=True)

    def body(k, carry):
        mvs, cvs = carry
        cs, slabs, midxs, avecs, cntvecs = [], [], [], [], []
        for n in range(G):
            c = cvs[n, 0]
            slab = hm_ref[n, c]
            h2 = H // 2
            cond = slab[:h2] >= slab[h2:]
            v1 = jnp.where(cond, slab[:h2], slab[h2:])
            i1 = jnp.where(cond, flat2[:h2], flat2[h2:])
            v2 = jnp.minimum(slab[:h2], slab[h2:])
            h = h2
            while h > 1:
                h2 = h // 2
                cond = ((v1[:h2] > v1[h2:])
                        | ((v1[:h2] == v1[h2:]) & (i1[:h2] <= i1[h2:])))
                nv2 = jnp.maximum(jnp.minimum(v1[:h2], v1[h2:]),
                                  jnp.where(cond, v2[:h2], v2[h2:]))
                i1 = jnp.where(cond, i1[:h2], i1[h2:])
                v1 = jnp.where(cond, v1[:h2], v1[h2:])
                v2 = nv2
                h = h2
            is_max = v1 == mvs[n:n + 1]
            midxs.append(jnp.where(is_max, i1, _BIG))
            avecs.append(jnp.where(is_max, v2, v1))
            cntvecs.append(is_max.astype(jnp.int32))
            cs.append(c)
            slabs.append(slab)
        posvs = jnp.min(jnp.concatenate(midxs, axis=0), axis=1,
                        keepdims=True)
        avs = jnp.max(jnp.concatenate(avecs, axis=0), axis=1,
                      keepdims=True)
        cnts = jnp.sum(jnp.concatenate(cntvecs, axis=0), axis=1,
                       keepdims=True)
        cmaxs = jnp.where(cnts >= 2, mvs, avs)

        m2 = m2_ref[...]
        m2m = jnp.where(laneG == cvs, jnp.float32(-2.0), m2)
        sec_v = jnp.max(m2m, axis=1, keepdims=True)
        sec_c = jnp.min(jnp.where(m2m == sec_v, laneG, _BIG), axis=1,
                        keepdims=True)

        for n in range(G):
            hm_ref[n, cs[n]] = jnp.where(flat2 == posvs[n:n + 1], neg,
                                         slabs[n])
        m2_ref[...] = jnp.where(laneG == cvs, cmaxs, m2)

        take_c = (cmaxs > sec_v) | ((cmaxs == sec_v) & (cvs <= sec_c))
        nmvs = jnp.where(take_c, cmaxs, sec_v)
        ncvs = jnp.where(take_c, cvs, sec_c)

        sel = laneG == k
        q_ref[0] = jnp.where(sel, mvs, q_ref[0])
        q_ref[1] = jnp.where(sel, cvs.astype(jnp.float32), q_ref[1])
        q_ref[2] = jnp.where(sel, posvs.astype(jnp.float32), q_ref[2])
        return (nmvs, ncvs)

    lax.fori_loop(0, K, body, (mvs0, cvs0), unroll=4)

    for n in range(G):
        score = q_ref[0, n:n + 1, :K]
        clsv = q_ref[1, n:n + 1, :K]
        posi = q_ref[2, n:n + 1, :].astype(jnp.int32)
        yi = posi // W
        xi = posi % W

        onehot_y = (lax.broadcasted_iota(jnp.int32, (H, 128), 0)
                    == jnp.broadcast_to(yi, (H, 128))).astype(jnp.float32)
        onehot_x = (lax.broadcasted_iota(jnp.int32, (W, 128), 0)
                    == jnp.broadcast_to(xi, (W, 128))).astype(jnp.float32)

        def gather2(arr):
            t = lax.dot_general(
                arr, onehot_y, (((0,), (0,)), ((), ())),
                precision=lax.Precision.HIGHEST,
                preferred_element_type=jnp.float32)
            return jnp.sum(t * onehot_x, axis=0, keepdims=True)

        g_reg0 = gather2(reg_ref[n, 0])
        g_reg1 = gather2(reg_ref[n, 1])
        g_wh0 = gather2(wh_ref[n, 0])
        g_wh1 = gather2(wh_ref[n, 1])

        ys = yi.astype(jnp.float32)[:, :K] + g_reg1[:, :K]
        xs = xi.astype(jnp.float32)[:, :K] + g_reg0[:, :K]
        wv = g_wh0[:, :K]
        hv = g_wh1[:, :K]
        x1 = (xs - wv * 0.5) * _DOWN_RATIO
        y1 = (ys - hv * 0.5) * _DOWN_RATIO
        x2 = (xs + wv * 0.5) * _DOWN_RATIO
        y2 = (ys + hv * 0.5) * _DOWN_RATIO
        boxes_ref[n] = jnp.concatenate([x1, y1, x2, y2], axis=0)
        scores_ref[n] = score
        cls_ref[n] = clsv


def kernel(hm, wh, reg):
    B, C, H, W = hm.shape
    K = _K
    G = _G
    body = functools.partial(_decode_body, C=C, H=H, W=W, K=K, G=G)
    boxes_t, scores, classes = pl.pallas_call(
        body,
        grid=(B // G,),
        in_specs=[
            pl.BlockSpec((G, C, H, W), lambda b: (b, 0, 0, 0)),
            pl.BlockSpec((G, 2, H, W), lambda b: (b, 0, 0, 0)),
            pl.BlockSpec((G, 2, H, W), lambda b: (b, 0, 0, 0)),
        ],
        out_specs=[
            pl.BlockSpec((G, 4, K), lambda b: (b, 0, 0)),
            pl.BlockSpec((G, 1, K), lambda b: (b, 0, 0)),
            pl.BlockSpec((G, 1, K), lambda b: (b, 0, 0)),
        ],
        out_shape=[
            jax.ShapeDtypeStruct((B, 4, K), jnp.float32),
            jax.ShapeDtypeStruct((B, 1, K), jnp.float32),
            jax.ShapeDtypeStruct((B, 1, K), jnp.float32),
        ],
        scratch_shapes=[
            pltpu.VMEM((G, 128), jnp.float32),
            pltpu.VMEM((3, G, 128), jnp.float32),
        ],
    )(hm, wh, reg)
    boxes = jnp.transpose(boxes_t, (0, 2, 1))
    return boxes, scores[:, 0, :], classes[:, 0, :]
